# Initial kernel scaffold; baseline (speedup 1.0000x reference)
import jax
import jax.numpy as jnp
from jax.experimental import pallas as pl


def kernel(x_1, x_2, n_1, n_2, edge_index, t1_W1, t1_W2, t2_W1, t2_W2, g_W1, g_W2, t1_b1, t1_b2, t2_b1, t2_b2, g_b1, g_b2):
    raise NotImplementedError("write your pallas kernel here")


# speedup vs baseline: 2.4101x; 2.4101x over previous
import functools

import jax
import jax.numpy as jnp
from jax import lax
from jax.experimental import pallas as pl
from jax.experimental.pallas import tpu as pltpu
from jax.experimental.pallas import tpu_sc as plsc

F32 = jnp.float32
_INV_S = 1.0 / (1.0 + 1e-5) ** 0.5
_MAXN = 1.0 - 1e-5
_PREC = lax.Precision.HIGHEST


def _rnorm(x):
    return jnp.maximum(jnp.sqrt(jnp.sum(x * x, axis=-1, keepdims=True)), 1e-15)


def _artanh(x):
    x = jnp.clip(x, -1.0 + 1e-7, 1.0 - 1e-7)
    return 0.5 * (jnp.log1p(x) - jnp.log1p(-x))


def _proj(x):
    n = _rnorm(x)
    return jnp.where(n > _MAXN, x / n * _MAXN, x)


def _expmap0(u):
    n = _rnorm(u)
    return jnp.tanh(n) * u / n


def _logmap0(p):
    n = _rnorm(p)
    return _artanh(n) * p / n


def _mobius_add(x, y):
    x2 = jnp.sum(x * x, -1, keepdims=True)
    y2 = jnp.sum(y * y, -1, keepdims=True)
    xy = jnp.sum(x * y, -1, keepdims=True)
    num = (1 + 2 * xy + y2) * x + (1 - x2) * y
    den = 1 + 2 * xy + x2 * y2
    return num / jnp.maximum(den, 1e-15)


def _matvecT(x, W):
    return lax.dot_general(x, W, (((1,), (1,)), ((), ())),
                           precision=_PREC, preferred_element_type=F32)


def _mobius_matvec_block(y, W):
    xn = _rnorm(y)
    mx = _matvecT(y, W)
    mn = _rnorm(mx)
    return jnp.tanh(mn / xn * _artanh(xn)) * mx / mn


def _stage_a_body(x_ref, w1_ref, b1_ref, w2_ref, b2_ref, gw_ref, gb_ref,
                  out_ref):
    x = x_ref[0]
    h = jax.nn.relu((_matvecT(x, w1_ref[0]) + b1_ref[0]) * _INV_S)
    h = jax.nn.relu((_matvecT(h, w2_ref[0]) + b2_ref[0]) * _INV_S)
    y = _proj(_expmap0(h))
    mv = _proj(_mobius_matvec_block(y, gw_ref[...]))
    gb = _proj(_expmap0(gb_ref[...]))
    h2 = _proj(_mobius_add(mv, gb))
    out_ref[...] = _logmap0(h2)


def _run_stage_a(Xs, TW1, Tb1, TW2, Tb2, gW, gb, NP):
    T, P, d = Xs.shape
    PB = P // 128
    grid = (T, PB)
    return pl.pallas_call(
        _stage_a_body,
        grid=grid,
        in_specs=[
            pl.BlockSpec((1, 128, d), lambda t, i: (t, i, 0)),
            pl.BlockSpec((1, d, d), lambda t, i: (t, 0, 0)),
            pl.BlockSpec((1, 1, d), lambda t, i: (t, 0, 0)),
            pl.BlockSpec((1, d, d), lambda t, i: (t, 0, 0)),
            pl.BlockSpec((1, 1, d), lambda t, i: (t, 0, 0)),
            pl.BlockSpec((d, d), lambda t, i: (0, 0)),
            pl.BlockSpec((1, d), lambda t, i: (0, 0)),
        ],
        out_specs=pl.BlockSpec((128, d), lambda t, i: (t * PB + i, 0)),
        out_shape=jax.ShapeDtypeStruct((NP, d), F32),
    )(Xs, TW1, Tb1, TW2, Tb2, gW, gb)


def _stage_d_body(p0_ref, p1_ref, xt_ref, gw_ref, gb_ref, out_ref):
    agg = p0_ref[0] + p1_ref[0] + xt_ref[...]
    h = _proj(_expmap0(agg))
    h = _proj(_expmap0(jax.nn.relu(_logmap0(h))))
    mv = _proj(_mobius_matvec_block(h, gw_ref[...]))
    gb = _proj(_expmap0(gb_ref[...]))
    out_ref[...] = _logmap0(_proj(_mobius_add(mv, gb)))


def _run_stage_d(parts, xt1, gW, gb):
    NP, d = xt1.shape
    grid = (NP // 128,)
    return pl.pallas_call(
        _stage_d_body,
        grid=grid,
        in_specs=[
            pl.BlockSpec((1, 128, d), lambda i: (0, i, 0)),
            pl.BlockSpec((1, 128, d), lambda i: (1, i, 0)),
            pl.BlockSpec((128, d), lambda i: (i, 0)),
            pl.BlockSpec((d, d), lambda i: (0, 0)),
            pl.BlockSpec((1, d), lambda i: (0, 0)),
        ],
        out_specs=pl.BlockSpec((128, d), lambda i: (i, 0)),
        out_shape=jax.ShapeDtypeStruct((NP, d), F32),
    )(parts, parts, xt1, gW, gb)


def _stage_f_body(p0_ref, p1_ref, xt_ref, sel_ref, o1_ref, o2_ref, acc_ref):
    i = pl.program_id(0)
    SEL = acc_ref.shape[0]
    B = o1_ref.shape[0]

    @pl.when(i == 0)
    def _():
        acc_ref[...] = jnp.zeros_like(acc_ref)

    blk = p0_ref[0] + p1_ref[0] + xt_ref[...]
    col = lax.broadcasted_iota(jnp.int32, (SEL, 128), 1) + i * 128
    oh = (sel_ref[...] == col).astype(F32)
    acc_ref[...] += lax.dot_general(oh, blk, (((1,), (0,)), ((), ())),
                                    precision=_PREC,
                                    preferred_element_type=F32)

    @pl.when(i == pl.num_programs(0) - 1)
    def _():
        agg = acc_ref[...]
        h = _proj(_expmap0(agg))
        h = _proj(_expmap0(jax.nn.relu(_logmap0(h))))
        res = _proj(_expmap0(_logmap0(h)))
        o1_ref[...] = res[:B]
        o2_ref[...] = res[B:2 * B]


def _run_stage_f(parts, xt2, sel_b, B):
    NP, d = xt2.shape
    SEL = sel_b.shape[0]
    grid = (NP // 128,)
    return pl.pallas_call(
        _stage_f_body,
        grid=grid,
        in_specs=[
            pl.BlockSpec((1, 128, d), lambda i: (0, i, 0)),
            pl.BlockSpec((1, 128, d), lambda i: (1, i, 0)),
            pl.BlockSpec((128, d), lambda i: (i, 0)),
            pl.BlockSpec((SEL, d), lambda i: (0, 0)),
        ],
        out_specs=[
            pl.BlockSpec((B, d), lambda i: (0, 0)),
            pl.BlockSpec((B, d), lambda i: (0, 0)),
        ],
        out_shape=[
            jax.ShapeDtypeStruct((B, d), F32),
            jax.ShapeDtypeStruct((B, d), F32),
        ],
        scratch_shapes=[pltpu.VMEM((SEL, d), F32)],
    )(parts, parts, xt2, sel_b)


def _sc_aggregate(xt, srcp, dstp, qp, zrows):
    NP, d = xt.shape
    EP = srcp.shape[0]
    CH = 128
    NCH = EP // CH
    info = plsc.get_sparse_core_info()
    NC, NS = info.num_cores, info.num_subcores
    NW = NC * NS
    CPW = -(-NCH // NW)
    RPS = NP // NS
    mesh = plsc.VectorSubcoreMesh(core_axis_name="c", subcore_axis_name="s")

    @functools.partial(
        pl.kernel, mesh=mesh,
        out_type=jax.ShapeDtypeStruct((NC, NP, d), F32),
        scratch_types=[
            pltpu.VMEM((CH,), jnp.int32),
            pltpu.VMEM((CH,), jnp.int32),
            pltpu.VMEM((CH,), jnp.int32),
            pltpu.VMEM((CH,), jnp.int32),
            pltpu.VMEM((CH, d), F32),
            pltpu.VMEM_SHARED((NP, d), F32),
            pltpu.SemaphoreType.DMA,
            pltpu.SemaphoreType.DMA,
        ],
    )
    def agg_kernel(xt_hbm, src_hbm, dst_hbm, qp_hbm, z_hbm, out_hbm,
                   sidx, didx, sidx2, didx2, rows, acc, gsem, hsem):
        c = lax.axis_index("c")
        s = lax.axis_index("s")
        wid = s * NC + c

        r0 = s * RPS
        nfull = RPS // CH
        for k in range(nfull):
            pltpu.sync_copy(z_hbm, acc.at[pl.ds(r0 + k * CH, CH)])
        rem = RPS - nfull * CH
        if rem:
            pltpu.sync_copy(z_hbm.at[pl.ds(0, rem)],
                            acc.at[pl.ds(r0 + nfull * CH, rem)])
        plsc.subcore_barrier()

        start = wid * CPW
        n_i = jnp.maximum(jnp.minimum(CPW, NCH - wid * CPW), 0)

        def body(i, carry):
            base = (start + i) * CH
            pltpu.sync_copy(src_hbm.at[pl.ds(base, CH)], sidx)
            pltpu.sync_copy(dst_hbm.at[pl.ds(base, CH)], didx)
            cp_s = pltpu.async_copy(qp_hbm.at[sidx], sidx2, hsem)
            cp_d = pltpu.async_copy(qp_hbm.at[didx], didx2, hsem)
            cp_s.wait()
            cp_d.wait()
            pltpu.async_copy(xt_hbm.at[sidx2], rows, gsem).wait()
            pltpu.sync_copy(rows, acc.at[didx2], add=True)
            return carry

        lax.fori_loop(0, n_i, body, 0)
        plsc.subcore_barrier()
        pltpu.sync_copy(acc.at[pl.ds(r0, RPS)],
                        out_hbm.at[c, pl.ds(r0, RPS)])

    return agg_kernel(xt, srcp, dstp, qp, zrows)


def kernel(x_1, x_2, n_1, n_2, edge_index, t1_W1, t1_W2, t2_W1, t2_W2,
           g_W1, g_W2, t1_b1, t1_b2, t2_b1, t2_b2, g_b1, g_b2):
    N1, d = x_1.shape
    N2 = x_2.shape[0]
    N = N1 + N2
    B = n_1.shape[0]
    P1 = -(-N1 // 128) * 128
    P2 = -(-N2 // 128) * 128
    NP = P1 + P2

    n1 = n_1.astype(jnp.int32)
    n2 = n_2.astype(jnp.int32)
    cum = jnp.cumsum(n1 + n2)
    zero = jnp.zeros((1,), jnp.int32)
    C0 = jnp.concatenate([zero, cum[:-1]])
    c1 = jnp.concatenate([zero, jnp.cumsum(n1)[:-1]])
    c2 = jnp.concatenate([zero, jnp.cumsum(n2)[:-1]])
    j = jnp.arange(N, dtype=jnp.int32)
    g = jnp.searchsorted(cum, j, side='right')
    within = j - C0[g]
    perm = jnp.where(within < n1[g], c1[g] + within,
                     N1 + c2[g] + within - n1[g]).astype(jnp.int32)
    qp = perm + jnp.where(perm >= N1, P1 - N1, 0).astype(jnp.int32)
    QPAD = -(-N // 16) * 16
    qp_pad = jnp.concatenate(
        [qp, jnp.full((QPAD - N,), NP - 1, jnp.int32)]) if QPAD != N else qp
    a_idx = C0
    b_idx = C0 + n1
    SEL = -(-(2 * B) // 128) * 128
    sel = jnp.concatenate([qp[a_idx], qp[b_idx],
                           jnp.zeros((SEL - 2 * B,), jnp.int32)])
    sel_b = jnp.broadcast_to(sel[:, None], (SEL, d))

    E = edge_index.shape[1]
    CH = 128
    EP = -(-E // CH) * CH
    src = edge_index[0].astype(jnp.int32)
    dst = edge_index[1].astype(jnp.int32)
    if EP != E:
        src = jnp.concatenate([src, jnp.zeros((EP - E,), jnp.int32)])
        dst = jnp.concatenate([dst, jnp.full((EP - E,), N, jnp.int32)])
        qp_pad = qp_pad.at[N].set(NP - 1) if QPAD != N else qp_pad

    zrows = jnp.zeros((CH, d), F32)

    Xs = jnp.stack([jnp.pad(x_1, ((0, P1 - N1), (0, 0))),
                    jnp.pad(x_2, ((0, P2 - N2), (0, 0)))])
    TW1 = jnp.stack([t1_W1, t2_W1])
    TW2 = jnp.stack([t1_W2, t2_W2])
    Tb1 = jnp.stack([t1_b1, t2_b1]).reshape(2, 1, d)
    Tb2 = jnp.stack([t1_b2, t2_b2]).reshape(2, 1, d)
    gb1 = g_b1.reshape(1, d)
    gb2 = g_b2.reshape(1, d)

    xt1 = _run_stage_a(Xs, TW1, Tb1, TW2, Tb2, g_W1, gb1, NP)
    parts1 = _sc_aggregate(xt1, src, dst, qp_pad, zrows)
    xt2 = _run_stage_d(parts1, xt1, g_W2, gb2)
    parts2 = _sc_aggregate(xt2, src, dst, qp_pad, zrows)
    o1, o2 = _run_stage_f(parts2, xt2, sel_b.astype(jnp.int32), B)
    return (o1, o2)
